# Initial kernel scaffold; baseline (speedup 1.0000x reference)
import jax
import jax.numpy as jnp
from jax.experimental import pallas as pl


def kernel(points_coords, points_labels):
    raise NotImplementedError("write your pallas kernel here")


# speedup vs baseline: 11.4591x; 11.4591x over previous
import functools

import jax
import jax.numpy as jnp
import numpy as np
from jax import lax
from jax.experimental import pallas as pl
from jax.experimental.pallas import tpu as pltpu
from jax.experimental.pallas import tpu_sc as plsc

RADIUS = 8
DD, HH, WW = 128, 128, 128
BS, NPTS = 8, 16
NWORKERS = 32
PLANES = BS * 2 * DD
PLANES_PER_W = PLANES // NWORKERS
PLANE_ELEMS = HH * WW
EPS = np.float32(np.finfo(np.float32).eps)
INV_2SIG2 = np.float32(0.125)


def _plane_pass(buf, z, ch, points, write_values):
    for vx, vy, vz, lbl in points:
        dz = z - vz
        hit = (lbl == (1 - ch)) & (dz >= -RADIUS) & (dz <= RADIUS)

        @pl.when(hit)
        def _(vx=vx, vy=vy, dz=dz):
            dz2 = dz * dz
            s0 = jnp.maximum(vx - RADIUS, 0) & -8
            s1 = jnp.minimum(s0 + 16, WW - 16)

            def dy_body(t, _):
                dy = t - RADIUS
                y = vy + dy
                r2zy = dz2 + dy * dy

                @pl.when((y >= 0) & (y < HH))
                def _():
                    for s in (s0, s1):
                        xi = s + jnp.arange(16, dtype=jnp.int32)
                        dx = xi - vx
                        m = (dx >= -RADIUS) & (dx <= RADIUS)
                        off = y * WW + s
                        old = buf[pl.ds(off, 16)]
                        if write_values:
                            r2 = (dx * dx + r2zy).astype(jnp.float32)
                            val = jnp.exp(r2 * -INV_2SIG2)
                            val = jnp.where(val < EPS, 0.0, val)
                            buf[pl.ds(off, 16)] = jnp.where(m, val, old)
                        else:
                            buf[pl.ds(off, 16)] = jnp.where(m, 0.0, old)

                return 0

            lax.fori_loop(0, 2 * RADIUS + 1, dy_body, 0)


def _clickmap_body(pts_hbm, out_hbm, pts_v, zbuf, sbuf):
    cid = lax.axis_index("c")
    sid = lax.axis_index("s")
    w = sid * 2 + cid
    b = w // 4
    ch = (w // 2) % 2
    z0 = (w % 2) * PLANES_PER_W
    base_p = w * PLANES_PER_W

    pltpu.sync_copy(pts_hbm.at[b], pts_v)

    zeros16 = jnp.zeros((16,), jnp.float32)

    def zero_all(i, _):
        zbuf[pl.ds(i * 16, 16)] = zeros16
        sbuf[pl.ds(i * 16, 16)] = zeros16
        return 0

    lax.fori_loop(0, PLANE_ELEMS // 16, zero_all, 0)

    vxs = pts_v[pl.ds(0, 16)]
    vys = pts_v[pl.ds(NPTS, 16)]
    vzs = pts_v[pl.ds(2 * NPTS, 16)]
    lbls = pts_v[pl.ds(3 * NPTS, 16)]
    points = [(vxs[j], vys[j], vzs[j], lbls[j]) for j in range(NPTS)]

    def plane_body(zi, _):
        z = z0 + zi
        p = base_p + zi
        dzv = z - vzs
        hitv = (lbls == (1 - ch)) & (dzv >= -RADIUS) & (dzv <= RADIUS)
        cnt = plsc.all_reduce_population_count(hitv)
        if cnt.ndim:
            cnt = cnt[0]
        has = cnt > 0

        @pl.when(jnp.logical_not(has))
        def _():
            pltpu.sync_copy(zbuf, out_hbm.at[p])

        @pl.when(has)
        def _():
            _plane_pass(sbuf, z, ch, points, write_values=True)
            pltpu.sync_copy(sbuf, out_hbm.at[p])
            _plane_pass(sbuf, z, ch, points, write_values=False)

        return 0

    lax.fori_loop(0, PLANES_PER_W, plane_body, 0)


@functools.partial(jax.jit, static_argnums=())
def _clickmap(pts):
    f = pl.kernel(
        _clickmap_body,
        out_type=jax.ShapeDtypeStruct((PLANES, PLANE_ELEMS), jnp.float32),
        mesh=plsc.VectorSubcoreMesh(core_axis_name="c", subcore_axis_name="s"),
        compiler_params=pltpu.CompilerParams(needs_layout_passes=False),
        scratch_types=[
            pltpu.VMEM((4 * NPTS,), jnp.int32),
            pltpu.VMEM((PLANE_ELEMS,), jnp.float32),
            pltpu.VMEM((PLANE_ELEMS,), jnp.float32),
        ],
    )
    return f(pts)


def kernel(points_coords, points_labels):
    coords = points_coords.astype(jnp.int32)
    labels = points_labels.astype(jnp.int32)
    pts = jnp.concatenate(
        [coords[:, :, 0], coords[:, :, 1], coords[:, :, 2], labels], axis=1
    )
    out = _clickmap(pts)
    return out.reshape(BS, 2, DD, HH, WW)
